# baseline (speedup 1.0000x reference)
import functools

import jax
import jax.numpy as jnp
from jax import lax
from jax.experimental import pallas as pl
from jax.experimental.pallas import tpu as pltpu
from jax.experimental.pallas import tpu_sc as plsc

NC = 2
NS = 16
NW = NC * NS
L = 16
CHUNK = 128


def _repack_body(x_ref, o_ref):
    d = x_ref.shape[1]
    o_ref[:, 0:d] = x_ref[...]
    o_ref[:, d:] = jnp.zeros_like(o_ref[:, d:])


@functools.lru_cache(maxsize=None)
def _make_fused(batch, du, dm, gd):
    bpw = batch // NW
    ngroups = bpw // L
    nchunks = bpw // CHUNK
    mesh = plsc.VectorSubcoreMesh(core_axis_name="c", subcore_axis_name="s")
    nw = du + dm + gd

    @functools.partial(
        pl.kernel,
        mesh=mesh,
        compiler_params=pltpu.CompilerParams(needs_layout_passes=False, use_tc_tiling_on_sc=True),
        out_type=jax.ShapeDtypeStruct((batch,), jnp.float32),
        scratch_types=[
            pltpu.VMEM((bpw,), jnp.int32),
            pltpu.VMEM((bpw,), jnp.int32),
            pltpu.VMEM((bpw, du), jnp.float32),
            pltpu.VMEM((CHUNK, 128), jnp.float32),
            pltpu.VMEM((bpw * du,), jnp.float32),
            pltpu.VMEM((bpw * gd,), jnp.float32),
            pltpu.VMEM(((nw + 1) * L,), jnp.float32),
            pltpu.VMEM((bpw,), jnp.float32),
            pltpu.SemaphoreType.DMA,
            pltpu.SemaphoreType.DMA,
            pltpu.SemaphoreType.DMA,
        ],
    )
    def fused_k(uid_hbm, mid_hbm, gflat_hbm, wq_hbm, utab_hbm, mpad_hbm,
                out_hbm, uidx_v, midx_v, ustage_v, mstage_v, flat_v, g_v,
                wq_v, out_v, usem, msem, gsem):
        wid = lax.axis_index("s") * NC + lax.axis_index("c")
        base = wid * bpw
        pltpu.sync_copy(uid_hbm.at[pl.ds(base, bpw)], uidx_v)
        pltpu.sync_copy(mid_hbm.at[pl.ds(base, bpw)], midx_v)
        gcp = pltpu.async_copy(gflat_hbm.at[pl.ds(base * gd, bpw * gd)],
                               g_v, gsem)

        def fire(g, carry):
            o = g * L
            uvec = uidx_v[pl.ds(o, L)]
            for t in range(L):
                pltpu.async_copy(
                    utab_hbm.at[pl.ds(uvec[t], 1)],
                    ustage_v.at[pl.ds(o + t, 1)], usem)
            return carry

        lax.fori_loop(0, ngroups, fire, 0)
        pltpu.sync_copy(wq_hbm, wq_v)

        for c in range(nchunks):
            pltpu.async_copy(
                mpad_hbm.at[midx_v.at[pl.ds(c * CHUNK, CHUNK)]],
                mstage_v, msem).wait()

            def mcompact(i, carry):
                fo = (c * CHUNK + i) * dm
                for h in range(dm // L):
                    flat_v[pl.ds(fo + h * L, L)] = \
                        mstage_v[i, pl.ds(h * L, L)]
                return carry

            lax.fori_loop(0, CHUNK, mcompact, 0)

        gcp.wait()
        lane = lax.iota(jnp.int32, L)
        bias = wq_v[pl.ds(nw * L, L)]

        def group1(g, carry):
            row = lane + g * L
            acc = bias
            mb = row * dm
            for j in range(dm):
                v = plsc.load_gather(flat_v, [mb + j])
                acc = acc + v * wq_v[pl.ds((du + j) * L, L)]
            gb = row * gd
            for j in range(gd):
                v = plsc.load_gather(g_v, [gb + j])
                acc = acc + v * wq_v[pl.ds((du + dm + j) * L, L)]
            out_v[pl.ds(g * L, L)] = acc
            return carry

        lax.fori_loop(0, ngroups, group1, 0)

        pltpu.make_async_copy(utab_hbm.at[pl.ds(0, bpw)], ustage_v,
                              usem).wait()

        def ucompact(i, carry):
            fo = i * du
            for h in range(du // L):
                flat_v[pl.ds(fo + h * L, L)] = ustage_v[i, pl.ds(h * L, L)]
            return carry

        lax.fori_loop(0, bpw, ucompact, 0)

        def group2(g, carry):
            row = lane + g * L
            acc = out_v[pl.ds(g * L, L)]
            ub = row * du
            for j in range(du):
                v = plsc.load_gather(flat_v, [ub + j])
                acc = acc + v * wq_v[pl.ds(j * L, L)]
            out_v[pl.ds(g * L, L)] = acc
            return carry

        lax.fori_loop(0, ngroups, group2, 0)
        pltpu.sync_copy(out_v, out_hbm.at[pl.ds(base, bpw)])

    return fused_k


def kernel(user_id, movie_id, genre_features, user_table, movie_table,
           genre_W, genre_b, fc_W, fc_b):
    batch = user_id.shape[0]
    du = user_table.shape[1]
    dm = movie_table.shape[1]
    gd = genre_features.shape[1]
    nm = movie_table.shape[0]

    uid = user_id.astype(jnp.int32)
    mid = movie_id.astype(jnp.int32)
    gflat = genre_features.reshape(-1)

    blk = 5000
    repack = pl.pallas_call(
        _repack_body,
        grid=(nm // blk,),
        in_specs=[pl.BlockSpec((blk, dm), lambda i: (i, 0))],
        out_specs=pl.BlockSpec((blk, 128), lambda i: (i, 0)),
        out_shape=jax.ShapeDtypeStruct((nm, 128), jnp.float32),
    )
    mpad = repack(movie_table)

    w = fc_W[0]
    wg = w[du + dm:]
    wg_eff = genre_W.T @ wg
    bias = fc_b[0] + genre_b @ wg
    wq = jnp.concatenate([
        jnp.repeat(w[:du + dm], L),
        jnp.repeat(wg_eff, L),
        jnp.full((L,), bias, jnp.float32),
    ])

    out = _make_fused(batch, du, dm, gd)(uid, mid, gflat, wq,
                                         user_table, mpad)
    return out.reshape(batch, 1)


# speedup vs baseline: 1.0945x; 1.0945x over previous
import functools

import jax
import jax.numpy as jnp
from jax import lax
from jax.experimental import pallas as pl
from jax.experimental.pallas import tpu as pltpu
from jax.experimental.pallas import tpu_sc as plsc

NC = 2
NS = 16
NW = NC * NS
L = 16
CHUNK = 128


@functools.lru_cache(maxsize=None)
def _make_fused(batch, du, dm, gd):
    bpw = batch // NW
    ngroups = bpw // L
    nchunks = bpw // CHUNK
    mesh = plsc.VectorSubcoreMesh(core_axis_name="c", subcore_axis_name="s")
    nw = du + dm + gd

    @functools.partial(
        pl.kernel,
        mesh=mesh,
        compiler_params=pltpu.CompilerParams(needs_layout_passes=False, use_tc_tiling_on_sc=True),
        out_type=jax.ShapeDtypeStruct((batch,), jnp.float32),
        scratch_types=[
            pltpu.VMEM((bpw,), jnp.int32),
            pltpu.VMEM((bpw,), jnp.int32),
            pltpu.VMEM((bpw, du), jnp.float32),
            pltpu.VMEM((CHUNK, 32), jnp.float32),
            pltpu.VMEM((bpw * du,), jnp.float32),
            pltpu.VMEM((bpw * gd,), jnp.float32),
            pltpu.VMEM(((nw + 1) * L,), jnp.float32),
            pltpu.VMEM((bpw,), jnp.float32),
            pltpu.SemaphoreType.DMA,
            pltpu.SemaphoreType.DMA,
            pltpu.SemaphoreType.DMA,
        ],
    )
    def fused_k(uid_hbm, mid_hbm, gflat_hbm, wq_hbm, utab_hbm, mtab_hbm,
                out_hbm, uidx_v, midx_v, ustage_v, mstage_v, flat_v, g_v,
                wq_v, out_v, usem, msem, gsem):
        wid = lax.axis_index("s") * NC + lax.axis_index("c")
        base = wid * bpw
        pltpu.sync_copy(uid_hbm.at[pl.ds(base, bpw)], uidx_v)
        pltpu.sync_copy(mid_hbm.at[pl.ds(base, bpw)], midx_v)
        gcp = pltpu.async_copy(gflat_hbm.at[pl.ds(base * gd, bpw * gd)],
                               g_v, gsem)

        def fire(g, carry):
            o = g * L
            uvec = uidx_v[pl.ds(o, L)]
            for t in range(L):
                pltpu.async_copy(
                    utab_hbm.at[pl.ds(uvec[t], 1)],
                    ustage_v.at[pl.ds(o + t, 1)], usem)
            return carry

        lax.fori_loop(0, ngroups, fire, 0)
        pltpu.sync_copy(wq_hbm, wq_v)

        for c in range(nchunks):
            def mfire(g, carry):
                o = c * CHUNK + g * L
                mvec = midx_v[pl.ds(o, L)]
                for t in range(L):
                    pltpu.async_copy(
                        mtab_hbm.at[pl.ds(mvec[t], 1)],
                        mstage_v.at[pl.ds(g * L + t, 1)], msem)
                return carry

            lax.fori_loop(0, CHUNK // L, mfire, 0)
            pltpu.make_async_copy(mtab_hbm.at[pl.ds(0, CHUNK)], mstage_v,
                                  msem).wait()

            def mcompact(i, carry):
                fo = (c * CHUNK + i) * dm
                for h in range(dm // L):
                    flat_v[pl.ds(fo + h * L, L)] = \
                        mstage_v[i, pl.ds(h * L, L)]
                return carry

            lax.fori_loop(0, CHUNK, mcompact, 0)

        gcp.wait()
        lane = lax.iota(jnp.int32, L)
        bias = wq_v[pl.ds(nw * L, L)]

        def group1(g, carry):
            row = lane + g * L
            acc = bias
            mb = row * dm
            for j in range(dm):
                v = plsc.load_gather(flat_v, [mb + j])
                acc = acc + v * wq_v[pl.ds((du + j) * L, L)]
            gb = row * gd
            for j in range(gd):
                v = plsc.load_gather(g_v, [gb + j])
                acc = acc + v * wq_v[pl.ds((du + dm + j) * L, L)]
            out_v[pl.ds(g * L, L)] = acc
            return carry

        lax.fori_loop(0, ngroups, group1, 0)

        pltpu.make_async_copy(utab_hbm.at[pl.ds(0, bpw)], ustage_v,
                              usem).wait()

        def ucompact(i, carry):
            fo = i * du
            for h in range(du // L):
                flat_v[pl.ds(fo + h * L, L)] = ustage_v[i, pl.ds(h * L, L)]
            return carry

        lax.fori_loop(0, bpw, ucompact, 0)

        def group2(g, carry):
            row = lane + g * L
            acc = out_v[pl.ds(g * L, L)]
            ub = row * du
            for j in range(du):
                v = plsc.load_gather(flat_v, [ub + j])
                acc = acc + v * wq_v[pl.ds(j * L, L)]
            out_v[pl.ds(g * L, L)] = acc
            return carry

        lax.fori_loop(0, ngroups, group2, 0)
        pltpu.sync_copy(out_v, out_hbm.at[pl.ds(base, bpw)])

    return fused_k


def kernel(user_id, movie_id, genre_features, user_table, movie_table,
           genre_W, genre_b, fc_W, fc_b):
    batch = user_id.shape[0]
    du = user_table.shape[1]
    dm = movie_table.shape[1]
    gd = genre_features.shape[1]
    nm = movie_table.shape[0]

    uid = user_id.astype(jnp.int32)
    mid = movie_id.astype(jnp.int32)
    gflat = genre_features.reshape(-1)

    w = fc_W[0]
    wg = w[du + dm:]
    wg_eff = genre_W.T @ wg
    bias = fc_b[0] + genre_b @ wg
    wq = jnp.concatenate([
        jnp.repeat(w[:du + dm], L),
        jnp.repeat(wg_eff, L),
        jnp.full((L,), bias, jnp.float32),
    ])

    out = _make_fused(batch, du, dm, gd)(uid, mid, gflat, wq,
                                         user_table, movie_table)
    return out.reshape(batch, 1)


# speedup vs baseline: 1.1431x; 1.0445x over previous
import functools

import jax
import jax.numpy as jnp
from jax import lax
from jax.experimental import pallas as pl
from jax.experimental.pallas import tpu as pltpu
from jax.experimental.pallas import tpu_sc as plsc

NC = 2
NS = 16
NW = NC * NS
L = 16
CHUNK = 128

_PARAMS = pltpu.CompilerParams(needs_layout_passes=False,
                               use_tc_tiling_on_sc=True)
_MESH = dict(core_axis_name="c", subcore_axis_name="s")


@functools.lru_cache(maxsize=None)
def _make_movie(batch, dm, gd, du):
    bpw = batch // NW
    ngroups = bpw // L
    nchunks = bpw // CHUNK
    nw = du + dm + gd

    @functools.partial(
        pl.kernel,
        mesh=plsc.VectorSubcoreMesh(**_MESH),
        compiler_params=_PARAMS,
        out_type=jax.ShapeDtypeStruct((batch,), jnp.float32),
        scratch_types=[
            pltpu.VMEM((bpw,), jnp.int32),
            pltpu.VMEM((CHUNK, 32), jnp.float32),
            pltpu.VMEM((bpw * dm,), jnp.float32),
            pltpu.VMEM((bpw * gd,), jnp.float32),
            pltpu.VMEM(((nw + 1) * L,), jnp.float32),
            pltpu.VMEM((bpw,), jnp.float32),
            pltpu.SemaphoreType.DMA,
            pltpu.SemaphoreType.DMA,
        ],
    )
    def movie_k(mid_hbm, gflat_hbm, wq_hbm, mtab_hbm, out_hbm,
                midx_v, mstage_v, flat_v, g_v, wq_v, out_v, msem, gsem):
        wid = lax.axis_index("s") * NC + lax.axis_index("c")
        base = wid * bpw
        pltpu.sync_copy(mid_hbm.at[pl.ds(base, bpw)], midx_v)
        gcp = pltpu.async_copy(gflat_hbm.at[pl.ds(base * gd, bpw * gd)],
                               g_v, gsem)
        pltpu.sync_copy(wq_hbm, wq_v)

        for c in range(nchunks):
            def mfire(g, carry):
                o = c * CHUNK + g * L
                mvec = midx_v[pl.ds(o, L)]
                for t in range(L):
                    pltpu.async_copy(
                        mtab_hbm.at[pl.ds(mvec[t], 1)],
                        mstage_v.at[pl.ds(g * L + t, 1)], msem)
                return carry

            lax.fori_loop(0, CHUNK // L, mfire, 0)
            pltpu.make_async_copy(mtab_hbm.at[pl.ds(0, CHUNK)], mstage_v,
                                  msem).wait()

            def mcompact(i, carry):
                fo = (c * CHUNK + i) * dm
                for h in range(dm // L):
                    flat_v[pl.ds(fo + h * L, L)] = \
                        mstage_v[i, pl.ds(h * L, L)]
                return carry

            lax.fori_loop(0, CHUNK, mcompact, 0)

        gcp.wait()
        lane = lax.iota(jnp.int32, L)
        bias = wq_v[pl.ds(nw * L, L)]

        def group1(g, carry):
            row = lane + g * L
            acc = bias
            mb = row * dm
            for j in range(dm):
                v = plsc.load_gather(flat_v, [mb + j])
                acc = acc + v * wq_v[pl.ds((du + j) * L, L)]
            gb = row * gd
            for j in range(gd):
                v = plsc.load_gather(g_v, [gb + j])
                acc = acc + v * wq_v[pl.ds((du + dm + j) * L, L)]
            out_v[pl.ds(g * L, L)] = acc
            return carry

        lax.fori_loop(0, ngroups, group1, 0)
        pltpu.sync_copy(out_v, out_hbm.at[pl.ds(base, bpw)])

    return movie_k


@functools.lru_cache(maxsize=None)
def _make_user(batch, du):
    bpw = batch // NW
    ngroups = bpw // L

    @functools.partial(
        pl.kernel,
        mesh=plsc.VectorSubcoreMesh(**_MESH),
        compiler_params=_PARAMS,
        out_type=jax.ShapeDtypeStruct((batch,), jnp.float32),
        scratch_types=[
            pltpu.VMEM((bpw,), jnp.int32),
            pltpu.VMEM((bpw, du), jnp.float32),
            pltpu.VMEM((bpw * du,), jnp.float32),
            pltpu.VMEM((du * L,), jnp.float32),
            pltpu.VMEM((bpw,), jnp.float32),
            pltpu.SemaphoreType.DMA,
            pltpu.SemaphoreType.DMA,
        ],
    )
    def user_k(uid_hbm, wqu_hbm, part_hbm, utab_hbm, out_hbm,
               uidx_v, ustage_v, flat_v, wq_v, out_v, usem, psem):
        wid = lax.axis_index("s") * NC + lax.axis_index("c")
        base = wid * bpw
        pltpu.sync_copy(uid_hbm.at[pl.ds(base, bpw)], uidx_v)

        def fire(g, carry):
            o = g * L
            uvec = uidx_v[pl.ds(o, L)]
            for t in range(L):
                pltpu.async_copy(
                    utab_hbm.at[pl.ds(uvec[t], 1)],
                    ustage_v.at[pl.ds(o + t, 1)], usem)
            return carry

        lax.fori_loop(0, ngroups, fire, 0)
        pcp = pltpu.async_copy(part_hbm.at[pl.ds(base, bpw)], out_v, psem)
        pltpu.sync_copy(wqu_hbm, wq_v)
        pltpu.make_async_copy(utab_hbm.at[pl.ds(0, bpw)], ustage_v,
                              usem).wait()

        def ucompact(i, carry):
            fo = i * du
            for h in range(du // L):
                flat_v[pl.ds(fo + h * L, L)] = ustage_v[i, pl.ds(h * L, L)]
            return carry

        lax.fori_loop(0, bpw, ucompact, 0)
        pcp.wait()
        lane = lax.iota(jnp.int32, L)

        def group2(g, carry):
            row = lane + g * L
            acc = out_v[pl.ds(g * L, L)]
            ub = row * du
            for j in range(du):
                v = plsc.load_gather(flat_v, [ub + j])
                acc = acc + v * wq_v[pl.ds(j * L, L)]
            out_v[pl.ds(g * L, L)] = acc
            return carry

        lax.fori_loop(0, ngroups, group2, 0)
        pltpu.sync_copy(out_v, out_hbm.at[pl.ds(base, bpw)])

    return user_k


def kernel(user_id, movie_id, genre_features, user_table, movie_table,
           genre_W, genre_b, fc_W, fc_b):
    batch = user_id.shape[0]
    du = user_table.shape[1]
    dm = movie_table.shape[1]
    gd = genre_features.shape[1]

    uid = user_id.astype(jnp.int32)
    mid = movie_id.astype(jnp.int32)
    gflat = genre_features.reshape(-1)

    w = fc_W[0]
    wg = w[du + dm:]
    wg_eff = genre_W.T @ wg
    bias = fc_b[0] + genre_b @ wg
    wq = jnp.concatenate([
        jnp.repeat(w[:du + dm], L),
        jnp.repeat(wg_eff, L),
        jnp.full((L,), bias, jnp.float32),
    ])
    wqu = jnp.repeat(w[:du], L)

    part = _make_movie(batch, dm, gd, du)(mid, gflat, wq, movie_table)
    out = _make_user(batch, du)(uid, wqu, part, user_table)
    return out.reshape(batch, 1)
